# baseline (speedup 1.0000x reference)
import jax
import jax.numpy as jnp
from jax import lax
from jax.experimental import pallas as pl
from jax.experimental.pallas import tpu as pltpu
from jax.experimental.pallas import tpu_sc as plsc

_N = 10000
_E = 320000
_D = 128
_FE = 4
_FE16 = 16
_G = 64
_OUT = 64

_NC = 2
_NS = 16
_NW = _NC * _NS
_B = 128
_RPC = _B // 128
_CPW = 80
_EP = _NW * _CPW * _B
_RPT = 640
_NPAD = _NS * _RPT


def _edges_body(ea_hbm, src_hbm, ze_hbm, eout_hbm, idx_s, erows, acce, ssem):
    cid = lax.axis_index("c")
    sid = lax.axis_index("s")
    wid = sid * _NC + cid
    r0 = sid * _RPT
    pltpu.sync_copy(ze_hbm.at[pl.ds(0, _B)], erows.at[0])
    for k in range(_RPT // _B):
        pltpu.sync_copy(erows.at[0], acce.at[pl.ds(r0 + k * _B, _B)])
    plsc.subcore_barrier()
    base = wid * (_CPW * _B)

    def ld(c, b):
        pltpu.sync_copy(src_hbm.at[pl.ds(base + c * _B, _B)], idx_s.at[b])
        pltpu.sync_copy(ea_hbm.at[pl.ds(base + c * _B, _B)], erows.at[b])

    def wait_s(b):
        pltpu.make_async_copy(ea_hbm.at[pl.ds(0, _B)], erows.at[b], ssem).wait()

    ld(0, 0)
    pltpu.async_copy(erows.at[0], acce.at[idx_s.at[0]], ssem, add=True)
    ld(1, 1)

    def step(c, carry):
        b = lax.rem(c, 2)
        wait_s(1 - b)
        pltpu.async_copy(erows.at[b], acce.at[idx_s.at[b]], ssem, add=True)
        ld(c + 1, 1 - b)
        return carry

    lax.fori_loop(1, _CPW - 1, step, 0)

    c = _CPW - 1
    wait_s(1 - c % 2)
    pltpu.async_copy(erows.at[c % 2], acce.at[idx_s.at[c % 2]], ssem, add=True)
    wait_s(c % 2)
    plsc.subcore_barrier()
    for k in range(_RPT // _B):
        pltpu.sync_copy(acce.at[pl.ds(r0 + k * _B, _B)], erows.at[0])
        pltpu.sync_copy(erows.at[0], eout_hbm.at[cid, pl.ds(r0 + k * _B, _B)])


def _spmm_body(h_hbm, src_hbm, dst_hbm, z_hbm, out_hbm,
               idx_s, idx_d, rows, acc, gsem, ssem):
    cid = lax.axis_index("c")
    sid = lax.axis_index("s")
    wid = sid * _NC + cid
    r0 = sid * _RPT
    pltpu.sync_copy(z_hbm.at[pl.ds(0, _B)], rows.at[0])
    for k in range(_RPT // _B):
        pltpu.sync_copy(rows.at[0], acc.at[pl.ds(r0 + k * _B, _B)])
    plsc.subcore_barrier()
    base = wid * (_CPW * _B)

    def ld_idx(c, slot):
        e0 = base + c * _B
        pltpu.sync_copy(src_hbm.at[pl.ds(e0, _B)], idx_s.at[slot])
        pltpu.sync_copy(dst_hbm.at[pl.ds(e0, _B)], idx_d.at[slot])

    def wait_g(b):
        pltpu.make_async_copy(h_hbm.at[pl.ds(0, _B)], rows.at[b], gsem).wait()

    def wait_s(b):
        pltpu.make_async_copy(h_hbm.at[pl.ds(0, _B)], rows.at[b], ssem).wait()

    ld_idx(0, 0)
    ld_idx(1, 1)
    pltpu.async_copy(h_hbm.at[idx_d.at[0]], rows.at[0], gsem)
    wait_g(0)
    pltpu.async_copy(h_hbm.at[idx_d.at[1]], rows.at[1], gsem)
    pltpu.async_copy(rows.at[0], acc.at[idx_s.at[0]], ssem, add=True)
    ld_idx(2, 2)

    def step(c, carry):
        b = lax.rem(c, 2)
        wait_g(b)
        wait_s(1 - b)
        pltpu.async_copy(h_hbm.at[idx_d.at[lax.rem(c + 1, 4)]],
                         rows.at[1 - b], gsem)
        pltpu.async_copy(rows.at[b], acc.at[idx_s.at[lax.rem(c, 4)]], ssem,
                         add=True)
        ld_idx(c + 2, lax.rem(c + 2, 4))
        return carry

    lax.fori_loop(1, _CPW - 2, step, 0)

    c = _CPW - 2
    wait_g(c % 2)
    wait_s(1 - c % 2)
    pltpu.async_copy(h_hbm.at[idx_d.at[(c + 1) % 4]], rows.at[1 - c % 2], gsem)
    pltpu.async_copy(rows.at[c % 2], acc.at[idx_s.at[c % 4]], ssem, add=True)
    c = _CPW - 1
    wait_g(c % 2)
    wait_s(1 - c % 2)
    pltpu.async_copy(rows.at[c % 2], acc.at[idx_s.at[c % 4]], ssem, add=True)
    wait_s(c % 2)
    plsc.subcore_barrier()
    nk = _RPT // _B
    pltpu.sync_copy(acc.at[pl.ds(r0, _B)], rows.at[0])
    pltpu.async_copy(rows.at[0], out_hbm.at[cid, pl.ds(r0, _B)], gsem)
    for k in range(1, nk):
        pltpu.sync_copy(acc.at[pl.ds(r0 + k * _B, _B)], rows.at[k % 2])
        pltpu.make_async_copy(h_hbm.at[pl.ds(0, _B)], rows.at[1 - k % 2], gsem).wait()
        pltpu.async_copy(rows.at[k % 2], out_hbm.at[cid, pl.ds(r0 + k * _B, _B)],
                         gsem)
    pltpu.make_async_copy(h_hbm.at[pl.ds(0, _B)], rows.at[(nk - 1) % 2], gsem).wait()


_sc_mesh = plsc.VectorSubcoreMesh(core_axis_name="c", subcore_axis_name="s",
                                  num_cores=_NC, num_subcores=_NS)

_edges = pl.kernel(
    _edges_body,
    out_type=[jax.ShapeDtypeStruct((_NC, _NPAD, _FE16), jnp.float32)],
    mesh=_sc_mesh,
    scratch_types=[
        pltpu.VMEM((2, _B), jnp.int32),
        pltpu.VMEM((2, _B, _FE16), jnp.float32),
        pltpu.VMEM_SHARED((_NPAD, _FE16), jnp.float32),
        pltpu.SemaphoreType.DMA,
    ],
    compiler_params=pltpu.CompilerParams(use_tc_tiling_on_sc=False),
    name="edges_sc",
)

_spmm = pl.kernel(
    _spmm_body,
    out_type=[jax.ShapeDtypeStruct((_NC, _NPAD, _D), jnp.float32)],
    mesh=_sc_mesh,
    scratch_types=[
        pltpu.VMEM((4, _B), jnp.int32),
        pltpu.VMEM((4, _B), jnp.int32),
        pltpu.VMEM((2, _B, _D), jnp.float32),
        pltpu.VMEM_SHARED((_NPAD, _D), jnp.float32),
        pltpu.SemaphoreType.DMA,
        pltpu.SemaphoreType.DMA,
    ],
    name="spmm_sc",
)


def _bn_relu(y, g, b):
    m = jnp.mean(y, axis=0, keepdims=True)
    d = y - m
    v = jnp.mean(d * d, axis=0, keepdims=True)
    return jnp.maximum(g * d * lax.rsqrt(v + 1e-5) + b, 0.0)


def _mlp_body(h_ref, s_ref, e_ref, w1a_ref, w1b_ref, b1_ref, g1_ref, be1_ref,
              w2_ref, b2_ref, go_ref, bo_ref, out_ref):
    agg = h_ref[...] + s_ref[0] + s_ref[1]
    er = e_ref[0] + e_ref[1] + 1.0
    y = (jnp.dot(agg, w1a_ref[...], preferred_element_type=jnp.float32, precision=lax.Precision.HIGHEST)
         + jnp.dot(er, w1b_ref[...], preferred_element_type=jnp.float32, precision=lax.Precision.HIGHEST)
         + b1_ref[...])
    y = _bn_relu(y, g1_ref[...], be1_ref[...])
    y = jnp.dot(y, w2_ref[...], preferred_element_type=jnp.float32, precision=lax.Precision.HIGHEST) + b2_ref[...]
    out_ref[...] = _bn_relu(y, go_ref[...], bo_ref[...])


_mlp = pl.pallas_call(
    _mlp_body,
    out_shape=jax.ShapeDtypeStruct((_N, _D), jnp.float32),
)


def _pool_body(ids_ref, x_ref, h1_ref, h2_ref, w0_ref, w1_ref, w2_ref,
               b0_ref, b1_ref, b2_ref, out_ref):
    ids = ids_ref[...]
    gi = lax.broadcasted_iota(jnp.int32, (_G, _N), 0)
    mask = jnp.where(gi == ids[None, :], 1.0, 0.0)
    p0 = jnp.dot(mask, x_ref[...], preferred_element_type=jnp.float32, precision=lax.Precision.HIGHEST)
    p1 = jnp.dot(mask, h1_ref[...], preferred_element_type=jnp.float32, precision=lax.Precision.HIGHEST)
    p2 = jnp.dot(mask, h2_ref[...], preferred_element_type=jnp.float32, precision=lax.Precision.HIGHEST)
    out_ref[...] = (jnp.dot(p0, w0_ref[...], preferred_element_type=jnp.float32, precision=lax.Precision.HIGHEST)
                    + jnp.dot(p1, w1_ref[...], preferred_element_type=jnp.float32, precision=lax.Precision.HIGHEST)
                    + jnp.dot(p2, w2_ref[...], preferred_element_type=jnp.float32, precision=lax.Precision.HIGHEST)
                    + b0_ref[...] + b1_ref[...] + b2_ref[...])


_pool = pl.pallas_call(
    _pool_body,
    out_shape=jax.ShapeDtypeStruct((_G, _OUT), jnp.float32),
)


def kernel(x, edge_index, edge_attr, batch_ids,
           l0_W1, l0_b1, l0_g1, l0_be1, l0_W2, l0_b2, l0_go, l0_bo,
           l1_W1, l1_b1, l1_g1, l1_be1, l1_W2, l1_b2, l1_go, l1_bo,
           p_W0, p_b0, p_W1, p_b1, p_W2, p_b2):
    pad = _EP - _E
    src = jnp.concatenate([edge_index[0], jnp.full((pad,), _N, jnp.int32)])
    dst = jnp.concatenate([edge_index[1], jnp.zeros((pad,), jnp.int32)])
    ea = jnp.pad(edge_attr, ((0, pad), (0, _FE16 - _FE)))
    z = jnp.zeros((_NPAD, _D), jnp.float32)
    ze = jnp.zeros((_NPAD, _FE16), jnp.float32)

    (e0,) = _edges(ea, src, ze)
    ev = e0[:, :_N]
    (s0,) = _spmm(x, src, dst, z)
    s0 = s0[:, :_N]
    w1a0, w1b0 = l0_W1[:_D], jnp.pad(l0_W1[_D:], ((0, _FE16 - _FE), (0, 0)))
    h1 = _mlp(x, s0, ev, w1a0, w1b0, l0_b1, l0_g1, l0_be1,
              l0_W2, l0_b2, l0_go, l0_bo)

    (s1,) = _spmm(h1, src, dst, z)
    s1 = s1[:, :_N]
    w1a1, w1b1 = l1_W1[:_D], jnp.pad(l1_W1[_D:], ((0, _FE16 - _FE), (0, 0)))
    h2 = _mlp(h1, s1, ev, w1a1, w1b1, l1_b1, l1_g1, l1_be1,
              l1_W2, l1_b2, l1_go, l1_bo)

    return _pool(batch_ids, x, h1, h2, p_W0, p_W1, p_W2, p_b0, p_b1, p_b2)


# speedup vs baseline: 2.0702x; 2.0702x over previous
import jax
import jax.numpy as jnp
from jax import lax
from jax.experimental import pallas as pl
from jax.experimental.pallas import tpu as pltpu
from jax.experimental.pallas import tpu_sc as plsc

_N = 10000
_E = 320000
_D = 128
_FE = 4
_FE16 = 16
_G = 64
_OUT = 64

_NC = 2
_NS = 16
_NW = _NC * _NS
_EPW = _E // _NW
_B = 128
_CPW = _EPW // _B
_TB = _EPW - _CPW * _B
_RPT = 640
_NPAD = _NS * _RPT


def _edges_body(ea_hbm, src_hbm, ze_hbm, eout_hbm, idx_s, erows, idx_t, erows_t,
                acce, ssem):
    cid = lax.axis_index("c")
    sid = lax.axis_index("s")
    wid = sid * _NC + cid
    r0 = sid * _RPT
    pltpu.sync_copy(ze_hbm, erows.at[0])
    for k in range(_RPT // _B):
        pltpu.sync_copy(erows.at[0], acce.at[pl.ds(r0 + k * _B, _B)])
    plsc.subcore_barrier()
    base = wid * _EPW

    def ld(c, b):
        pltpu.sync_copy(src_hbm.at[pl.ds(base + c * _B, _B)], idx_s.at[b])
        pltpu.sync_copy(ea_hbm.at[pl.ds(base + c * _B, _B)], erows.at[b])

    def wait_s(b):
        pltpu.make_async_copy(ea_hbm.at[pl.ds(0, _B)], erows.at[b], ssem).wait()

    ld(0, 0)
    pltpu.async_copy(erows.at[0], acce.at[idx_s.at[0]], ssem, add=True)
    ld(1, 1)

    def step(c, carry):
        b = lax.rem(c, 2)
        wait_s(1 - b)
        pltpu.async_copy(erows.at[b], acce.at[idx_s.at[b]], ssem, add=True)
        ld(c + 1, 1 - b)
        return carry

    lax.fori_loop(1, _CPW - 1, step, 0)

    c = _CPW - 1
    wait_s(1 - c % 2)
    pltpu.async_copy(erows.at[c % 2], acce.at[idx_s.at[c % 2]], ssem, add=True)
    wait_s(c % 2)
    t0 = base + _CPW * _B
    pltpu.sync_copy(src_hbm.at[pl.ds(t0, _TB)], idx_t)
    pltpu.sync_copy(ea_hbm.at[pl.ds(t0, _TB)], erows_t)
    pltpu.sync_copy(erows_t, acce.at[idx_t], add=True)
    plsc.subcore_barrier()
    for k in range(_RPT // _B):
        pltpu.sync_copy(acce.at[pl.ds(r0 + k * _B, _B)], erows.at[0])
        pltpu.sync_copy(erows.at[0], eout_hbm.at[cid, pl.ds(r0 + k * _B, _B)])


def _spmm_body(h_hbm, src_hbm, dst_hbm, z_hbm, out_hbm,
               idx_s, idx_d, rows, idx_st, idx_dt, rows_t, acc, gsem, ssem):
    cid = lax.axis_index("c")
    sid = lax.axis_index("s")
    wid = sid * _NC + cid
    r0 = sid * _RPT
    pltpu.sync_copy(z_hbm, rows.at[0])
    for k in range(_RPT // _B):
        pltpu.sync_copy(rows.at[0], acc.at[pl.ds(r0 + k * _B, _B)])
    plsc.subcore_barrier()
    base = wid * _EPW

    def ld_idx(c, slot):
        e0 = base + c * _B
        pltpu.sync_copy(src_hbm.at[pl.ds(e0, _B)], idx_s.at[slot])
        pltpu.sync_copy(dst_hbm.at[pl.ds(e0, _B)], idx_d.at[slot])

    def wait_g(b):
        pltpu.make_async_copy(h_hbm.at[pl.ds(0, _B)], rows.at[b], gsem).wait()

    def wait_s(b):
        pltpu.make_async_copy(h_hbm.at[pl.ds(0, _B)], rows.at[b], ssem).wait()

    ld_idx(0, 0)
    ld_idx(1, 1)
    pltpu.async_copy(h_hbm.at[idx_d.at[0]], rows.at[0], gsem)
    wait_g(0)
    pltpu.async_copy(h_hbm.at[idx_d.at[1]], rows.at[1], gsem)
    pltpu.async_copy(rows.at[0], acc.at[idx_s.at[0]], ssem, add=True)
    ld_idx(2, 2)

    def step(c, carry):
        b = lax.rem(c, 2)
        wait_g(b)
        wait_s(1 - b)
        pltpu.async_copy(h_hbm.at[idx_d.at[lax.rem(c + 1, 4)]],
                         rows.at[1 - b], gsem)
        pltpu.async_copy(rows.at[b], acc.at[idx_s.at[lax.rem(c, 4)]], ssem,
                         add=True)
        ld_idx(c + 2, lax.rem(c + 2, 4))
        return carry

    lax.fori_loop(1, _CPW - 2, step, 0)

    c = _CPW - 2
    wait_g(c % 2)
    wait_s(1 - c % 2)
    pltpu.async_copy(h_hbm.at[idx_d.at[(c + 1) % 4]], rows.at[1 - c % 2], gsem)
    pltpu.async_copy(rows.at[c % 2], acc.at[idx_s.at[c % 4]], ssem, add=True)
    c = _CPW - 1
    wait_g(c % 2)
    wait_s(1 - c % 2)
    pltpu.async_copy(rows.at[c % 2], acc.at[idx_s.at[c % 4]], ssem, add=True)
    wait_s(c % 2)
    t0 = base + _CPW * _B
    pltpu.sync_copy(src_hbm.at[pl.ds(t0, _TB)], idx_st)
    pltpu.sync_copy(dst_hbm.at[pl.ds(t0, _TB)], idx_dt)
    pltpu.async_copy(h_hbm.at[idx_dt], rows_t, gsem).wait()
    pltpu.sync_copy(rows_t, acc.at[idx_st], add=True)
    plsc.subcore_barrier()
    nk = _RPT // _B
    pltpu.sync_copy(acc.at[pl.ds(r0, _B)], rows.at[0])
    pltpu.async_copy(rows.at[0], out_hbm.at[cid, pl.ds(r0, _B)], gsem)
    for k in range(1, nk):
        pltpu.sync_copy(acc.at[pl.ds(r0 + k * _B, _B)], rows.at[k % 2])
        pltpu.make_async_copy(h_hbm.at[pl.ds(0, _B)], rows.at[1 - k % 2], gsem).wait()
        pltpu.async_copy(rows.at[k % 2], out_hbm.at[cid, pl.ds(r0 + k * _B, _B)],
                         gsem)
    pltpu.make_async_copy(h_hbm.at[pl.ds(0, _B)], rows.at[(nk - 1) % 2], gsem).wait()


_sc_mesh = plsc.VectorSubcoreMesh(core_axis_name="c", subcore_axis_name="s",
                                  num_cores=_NC, num_subcores=_NS)

_edges = pl.kernel(
    _edges_body,
    out_type=[jax.ShapeDtypeStruct((_NC, _NPAD, _FE16), jnp.float32)],
    mesh=_sc_mesh,
    scratch_types=[
        pltpu.VMEM((2, _B), jnp.int32),
        pltpu.VMEM((2, _B, _FE16), jnp.float32),
        pltpu.VMEM((_TB,), jnp.int32),
        pltpu.VMEM((_TB, _FE16), jnp.float32),
        pltpu.VMEM_SHARED((_NPAD, _FE16), jnp.float32),
        pltpu.SemaphoreType.DMA,
    ],
    compiler_params=pltpu.CompilerParams(use_tc_tiling_on_sc=False),
    name="edges_sc",
)

_spmm = pl.kernel(
    _spmm_body,
    out_type=[jax.ShapeDtypeStruct((_NC, _NPAD, _D), jnp.float32)],
    mesh=_sc_mesh,
    scratch_types=[
        pltpu.VMEM((4, _B), jnp.int32),
        pltpu.VMEM((4, _B), jnp.int32),
        pltpu.VMEM((2, _B, _D), jnp.float32),
        pltpu.VMEM((_TB,), jnp.int32),
        pltpu.VMEM((_TB,), jnp.int32),
        pltpu.VMEM((_TB, _D), jnp.float32),
        pltpu.VMEM_SHARED((_NPAD, _D), jnp.float32),
        pltpu.SemaphoreType.DMA,
        pltpu.SemaphoreType.DMA,
    ],
    name="spmm_sc",
)


def _bn_relu(y, g, b):
    m = jnp.mean(y, axis=0, keepdims=True)
    d = y - m
    v = jnp.mean(d * d, axis=0, keepdims=True)
    return jnp.maximum(g * d * lax.rsqrt(v + 1e-5) + b, 0.0)


def _mlp_body(h_ref, s_ref, e_ref, w1a_ref, w1b_ref, b1_ref, g1_ref, be1_ref,
              w2_ref, b2_ref, go_ref, bo_ref, out_ref):
    agg = h_ref[...] + s_ref[0] + s_ref[1]
    er = e_ref[0] + e_ref[1] + 1.0
    y = (jnp.dot(agg, w1a_ref[...], preferred_element_type=jnp.float32,
                 precision=lax.Precision.HIGHEST)
         + jnp.dot(er, w1b_ref[...], preferred_element_type=jnp.float32,
                   precision=lax.Precision.HIGHEST)
         + b1_ref[...])
    y = _bn_relu(y, g1_ref[...], be1_ref[...])
    y = jnp.dot(y, w2_ref[...], preferred_element_type=jnp.float32,
                precision=lax.Precision.HIGHEST) + b2_ref[...]
    out_ref[...] = _bn_relu(y, go_ref[...], bo_ref[...])


_mlp = pl.pallas_call(
    _mlp_body,
    out_shape=jax.ShapeDtypeStruct((_N, _D), jnp.float32),
)


def _pool_body(ids_ref, x_ref, h1_ref, h2_ref, w0_ref, w1_ref, w2_ref,
               b0_ref, b1_ref, b2_ref, out_ref):
    ids = ids_ref[...]
    gi = lax.broadcasted_iota(jnp.int32, (_G, _N), 0)
    mask = jnp.where(gi == ids[None, :], 1.0, 0.0)
    p0 = jnp.dot(mask, x_ref[...], preferred_element_type=jnp.float32,
                 precision=lax.Precision.HIGHEST)
    p1 = jnp.dot(mask, h1_ref[...], preferred_element_type=jnp.float32,
                 precision=lax.Precision.HIGHEST)
    p2 = jnp.dot(mask, h2_ref[...], preferred_element_type=jnp.float32,
                 precision=lax.Precision.HIGHEST)
    out_ref[...] = (
        jnp.dot(p0, w0_ref[...], preferred_element_type=jnp.float32,
                precision=lax.Precision.HIGHEST)
        + jnp.dot(p1, w1_ref[...], preferred_element_type=jnp.float32,
                  precision=lax.Precision.HIGHEST)
        + jnp.dot(p2, w2_ref[...], preferred_element_type=jnp.float32,
                  precision=lax.Precision.HIGHEST)
        + b0_ref[...] + b1_ref[...] + b2_ref[...])


_pool = pl.pallas_call(
    _pool_body,
    out_shape=jax.ShapeDtypeStruct((_G, _OUT), jnp.float32),
)


def kernel(x, edge_index, edge_attr, batch_ids,
           l0_W1, l0_b1, l0_g1, l0_be1, l0_W2, l0_b2, l0_go, l0_bo,
           l1_W1, l1_b1, l1_g1, l1_be1, l1_W2, l1_b2, l1_go, l1_bo,
           p_W0, p_b0, p_W1, p_b1, p_W2, p_b2):
    z = jnp.zeros((_B, _D), jnp.float32)
    ze = jnp.zeros((_B, _FE16), jnp.float32)
    src = edge_index[0]
    dst = edge_index[1]
    ea16 = jnp.pad(edge_attr, ((0, 0), (0, _FE16 - _FE)))

    (e0,) = _edges(ea16, src, ze)
    ev = e0[:, :_N]
    (s0,) = _spmm(x, src, dst, z)
    s0 = s0[:, :_N]
    w1b0 = jnp.pad(l0_W1[_D:], ((0, _FE16 - _FE), (0, 0)))
    h1 = _mlp(x, s0, ev, l0_W1[:_D], w1b0, l0_b1, l0_g1, l0_be1,
              l0_W2, l0_b2, l0_go, l0_bo)

    (s1,) = _spmm(h1, src, dst, z)
    s1 = s1[:, :_N]
    w1b1 = jnp.pad(l1_W1[_D:], ((0, _FE16 - _FE), (0, 0)))
    h2 = _mlp(h1, s1, ev, l1_W1[:_D], w1b1, l1_b1, l1_g1, l1_be1,
              l1_W2, l1_b2, l1_go, l1_bo)

    return _pool(batch_ids, x, h1, h2, p_W0, p_W1, p_W2, p_b0, p_b1, p_b2)


# speedup vs baseline: 2.1086x; 1.0186x over previous
import jax
import jax.numpy as jnp
from jax import lax
from jax.experimental import pallas as pl
from jax.experimental.pallas import tpu as pltpu
from jax.experimental.pallas import tpu_sc as plsc

_N = 10000
_E = 320000
_D = 128
_FE = 4
_FE16 = 16
_G = 64
_OUT = 64

_NC = 2
_NS = 16
_NW = _NC * _NS
_EPW = _E // _NW
_B = 128
_CPW = _EPW // _B
_TB = _EPW - _CPW * _B
_RPT = 640
_NPAD = _NS * _RPT


def _edges_body(ea_hbm, src_hbm, ze_hbm, eout_hbm, idx_s, erows, idx_t, erows_t,
                acce, ssem):
    cid = lax.axis_index("c")
    sid = lax.axis_index("s")
    wid = sid * _NC + cid
    r0 = sid * _RPT
    pltpu.sync_copy(ze_hbm, erows.at[0])
    for k in range(_RPT // _B):
        pltpu.sync_copy(erows.at[0], acce.at[pl.ds(r0 + k * _B, _B)])
    plsc.subcore_barrier()
    base = wid * _EPW

    def ld(c, b):
        pltpu.sync_copy(src_hbm.at[pl.ds(base + c * _B, _B)], idx_s.at[b])
        pltpu.sync_copy(ea_hbm.at[pl.ds(base + c * _B, _B)], erows.at[b])

    def wait_s(b):
        pltpu.make_async_copy(ea_hbm.at[pl.ds(0, _B)], erows.at[b], ssem).wait()

    ld(0, 0)
    pltpu.async_copy(erows.at[0], acce.at[idx_s.at[0]], ssem, add=True)
    ld(1, 1)

    def step(c, carry):
        b = lax.rem(c, 2)
        wait_s(1 - b)
        pltpu.async_copy(erows.at[b], acce.at[idx_s.at[b]], ssem, add=True)
        ld(c + 1, 1 - b)
        return carry

    lax.fori_loop(1, _CPW - 1, step, 0)

    c = _CPW - 1
    wait_s(1 - c % 2)
    pltpu.async_copy(erows.at[c % 2], acce.at[idx_s.at[c % 2]], ssem, add=True)
    wait_s(c % 2)
    t0 = base + _CPW * _B
    pltpu.sync_copy(src_hbm.at[pl.ds(t0, _TB)], idx_t)
    pltpu.sync_copy(ea_hbm.at[pl.ds(t0, _TB)], erows_t)
    pltpu.sync_copy(erows_t, acce.at[idx_t], add=True)
    plsc.subcore_barrier()
    for k in range(_RPT // _B):
        pltpu.sync_copy(acce.at[pl.ds(r0 + k * _B, _B)], erows.at[0])
        pltpu.sync_copy(erows.at[0], eout_hbm.at[cid, pl.ds(r0 + k * _B, _B)])


def _spmm_body(h_hbm, src_hbm, dst_hbm, z_hbm, out_hbm,
               idx_s, idx_d, rows, idx_st, idx_dt, rows_t, acc, gsem, ssem):
    cid = lax.axis_index("c")
    sid = lax.axis_index("s")
    wid = sid * _NC + cid
    r0 = sid * _RPT
    pltpu.sync_copy(z_hbm, rows.at[0])
    for k in range(_RPT // _B):
        pltpu.sync_copy(rows.at[0], acc.at[pl.ds(r0 + k * _B, _B)])
    plsc.subcore_barrier()
    base = wid * _EPW

    def ld_idx(c, slot):
        e0 = base + c * _B
        pltpu.sync_copy(src_hbm.at[pl.ds(e0, _B)], idx_s.at[slot])
        pltpu.sync_copy(dst_hbm.at[pl.ds(e0, _B)], idx_d.at[slot])

    def wait_g(b):
        pltpu.make_async_copy(h_hbm.at[pl.ds(0, _B)], rows.at[b], gsem).wait()

    def wait_s(b):
        pltpu.make_async_copy(h_hbm.at[pl.ds(0, _B)], rows.at[b], ssem).wait()

    ld_idx(0, 0)
    ld_idx(1, 1)
    pltpu.async_copy(h_hbm.at[idx_d.at[0]], rows.at[0], gsem)
    wait_g(0)
    pltpu.async_copy(h_hbm.at[idx_d.at[1]], rows.at[1], gsem)
    pltpu.async_copy(rows.at[0], acc.at[idx_s.at[0]], ssem, add=True)
    ld_idx(2, 2)

    def step(c, carry):
        b = lax.rem(c, 2)
        wait_g(b)
        wait_s(1 - b)
        pltpu.async_copy(h_hbm.at[idx_d.at[lax.rem(c + 1, 4)]],
                         rows.at[1 - b], gsem)
        pltpu.async_copy(rows.at[b], acc.at[idx_s.at[lax.rem(c, 4)]], ssem,
                         add=True)
        ld_idx(c + 2, lax.rem(c + 2, 4))
        return carry

    lax.fori_loop(1, _CPW - 2, step, 0)

    c = _CPW - 2
    wait_g(c % 2)
    wait_s(1 - c % 2)
    pltpu.async_copy(h_hbm.at[idx_d.at[(c + 1) % 4]], rows.at[1 - c % 2], gsem)
    pltpu.async_copy(rows.at[c % 2], acc.at[idx_s.at[c % 4]], ssem, add=True)
    c = _CPW - 1
    wait_g(c % 2)
    wait_s(1 - c % 2)
    pltpu.async_copy(rows.at[c % 2], acc.at[idx_s.at[c % 4]], ssem, add=True)
    wait_s(c % 2)
    t0 = base + _CPW * _B
    pltpu.sync_copy(src_hbm.at[pl.ds(t0, _TB)], idx_st)
    pltpu.sync_copy(dst_hbm.at[pl.ds(t0, _TB)], idx_dt)
    pltpu.async_copy(h_hbm.at[idx_dt], rows_t, gsem).wait()
    pltpu.sync_copy(rows_t, acc.at[idx_st], add=True)
    plsc.subcore_barrier()
    nk = _RPT // _B
    pltpu.sync_copy(acc.at[pl.ds(r0, _B)], rows.at[0])
    pltpu.async_copy(rows.at[0], out_hbm.at[cid, pl.ds(r0, _B)], gsem)
    for k in range(1, nk):
        pltpu.sync_copy(acc.at[pl.ds(r0 + k * _B, _B)], rows.at[k % 2])
        pltpu.make_async_copy(h_hbm.at[pl.ds(0, _B)], rows.at[1 - k % 2], gsem).wait()
        pltpu.async_copy(rows.at[k % 2], out_hbm.at[cid, pl.ds(r0 + k * _B, _B)],
                         gsem)
    pltpu.make_async_copy(h_hbm.at[pl.ds(0, _B)], rows.at[(nk - 1) % 2], gsem).wait()


_sc_mesh = plsc.VectorSubcoreMesh(core_axis_name="c", subcore_axis_name="s",
                                  num_cores=_NC, num_subcores=_NS)

_edges = pl.kernel(
    _edges_body,
    out_type=[jax.ShapeDtypeStruct((_NC, _NPAD, _FE16), jnp.float32)],
    mesh=_sc_mesh,
    scratch_types=[
        pltpu.VMEM((2, _B), jnp.int32),
        pltpu.VMEM((2, _B, _FE16), jnp.float32),
        pltpu.VMEM((_TB,), jnp.int32),
        pltpu.VMEM((_TB, _FE16), jnp.float32),
        pltpu.VMEM_SHARED((_NPAD, _FE16), jnp.float32),
        pltpu.SemaphoreType.DMA,
    ],
    compiler_params=pltpu.CompilerParams(use_tc_tiling_on_sc=False),
    name="edges_sc",
)

_spmm = pl.kernel(
    _spmm_body,
    out_type=[jax.ShapeDtypeStruct((_NC, _NPAD, _D), jnp.float32)],
    mesh=_sc_mesh,
    scratch_types=[
        pltpu.VMEM((4, _B), jnp.int32),
        pltpu.VMEM((4, _B), jnp.int32),
        pltpu.VMEM((2, _B, _D), jnp.float32),
        pltpu.VMEM((_TB,), jnp.int32),
        pltpu.VMEM((_TB,), jnp.int32),
        pltpu.VMEM((_TB, _D), jnp.float32),
        pltpu.VMEM_SHARED((_NPAD, _D), jnp.float32),
        pltpu.SemaphoreType.DMA,
        pltpu.SemaphoreType.DMA,
    ],
    name="spmm_sc",
)


def _bn_relu(y, g, b):
    m = jnp.mean(y, axis=0, keepdims=True)
    d = y - m
    v = jnp.mean(d * d, axis=0, keepdims=True)
    return jnp.maximum(g * d * lax.rsqrt(v + 1e-5) + b, 0.0)


def _mlp_body(h_ref, s_ref, e_ref, w1a_ref, w1b_ref, b1_ref, g1_ref, be1_ref,
              w2_ref, b2_ref, go_ref, bo_ref, out_ref):
    agg = h_ref[...] + s_ref[0, :_N] + s_ref[1, :_N]
    er = e_ref[0, :_N] + e_ref[1, :_N] + 1.0
    y = (jnp.dot(agg, w1a_ref[...], preferred_element_type=jnp.float32,
                 precision=lax.Precision.HIGHEST)
         + jnp.dot(er, w1b_ref[...], preferred_element_type=jnp.float32,
                   precision=lax.Precision.HIGHEST)
         + b1_ref[...])
    y = _bn_relu(y, g1_ref[...], be1_ref[...])
    y = jnp.dot(y, w2_ref[...], preferred_element_type=jnp.float32,
                precision=lax.Precision.HIGHEST) + b2_ref[...]
    out_ref[...] = _bn_relu(y, go_ref[...], bo_ref[...])


_mlp = pl.pallas_call(
    _mlp_body,
    out_shape=jax.ShapeDtypeStruct((_N, _D), jnp.float32),
)


def _pool_body(ids_ref, x_ref, h1_ref, h2_ref, w0_ref, w1_ref, w2_ref,
               b0_ref, b1_ref, b2_ref, out_ref):
    ids = ids_ref[...]
    gi = lax.broadcasted_iota(jnp.int32, (_G, _N), 0)
    mask = jnp.where(gi == ids[None, :], 1.0, 0.0)
    p0 = jnp.dot(mask, x_ref[...], preferred_element_type=jnp.float32,
                 precision=lax.Precision.HIGHEST)
    p1 = jnp.dot(mask, h1_ref[...], preferred_element_type=jnp.float32,
                 precision=lax.Precision.HIGHEST)
    p2 = jnp.dot(mask, h2_ref[...], preferred_element_type=jnp.float32,
                 precision=lax.Precision.HIGHEST)
    out_ref[...] = (
        jnp.dot(p0, w0_ref[...], preferred_element_type=jnp.float32,
                precision=lax.Precision.HIGHEST)
        + jnp.dot(p1, w1_ref[...], preferred_element_type=jnp.float32,
                  precision=lax.Precision.HIGHEST)
        + jnp.dot(p2, w2_ref[...], preferred_element_type=jnp.float32,
                  precision=lax.Precision.HIGHEST)
        + b0_ref[...] + b1_ref[...] + b2_ref[...])


_pool = pl.pallas_call(
    _pool_body,
    out_shape=jax.ShapeDtypeStruct((_G, _OUT), jnp.float32),
)


def kernel(x, edge_index, edge_attr, batch_ids,
           l0_W1, l0_b1, l0_g1, l0_be1, l0_W2, l0_b2, l0_go, l0_bo,
           l1_W1, l1_b1, l1_g1, l1_be1, l1_W2, l1_b2, l1_go, l1_bo,
           p_W0, p_b0, p_W1, p_b1, p_W2, p_b2):
    z = jnp.zeros((_B, _D), jnp.float32)
    ze = jnp.zeros((_B, _FE16), jnp.float32)
    src = edge_index[0]
    dst = edge_index[1]
    ea16 = jnp.pad(edge_attr, ((0, 0), (0, _FE16 - _FE)))

    (ev,) = _edges(ea16, src, ze)
    (s0,) = _spmm(x, src, dst, z)
    w1b0 = jnp.pad(l0_W1[_D:], ((0, _FE16 - _FE), (0, 0)))
    h1 = _mlp(x, s0, ev, l0_W1[:_D], w1b0, l0_b1, l0_g1, l0_be1,
              l0_W2, l0_b2, l0_go, l0_bo)

    (s1,) = _spmm(h1, src, dst, z)
    w1b1 = jnp.pad(l1_W1[_D:], ((0, _FE16 - _FE), (0, 0)))
    h2 = _mlp(h1, s1, ev, l1_W1[:_D], w1b1, l1_b1, l1_g1, l1_be1,
              l1_W2, l1_b2, l1_go, l1_bo)

    return _pool(batch_ids, x, h1, h2, p_W0, p_W1, p_W2, p_b0, p_b1, p_b2)
